# baseline (speedup 1.0000x reference)
import dataclasses
import functools

import jax
import jax.numpy as jnp
from jax import lax
from jax.experimental import pallas as pl
from jax.experimental.pallas import tpu as pltpu
from jax.experimental.pallas import tpu_sc as plsc

_N = 50000
_E = 800000
_NCORE = 2
_NSUB = 16
_LANES = 16
_W = 128
_ROWS = 6400
_EPAD = _ROWS * _W
_RPTI = _ROWS // _NSUB
_EPT = _RPTI * _W
_NPAD = 50048
_RPT = _NPAD // _NSUB
_ZR = 184


def _make_edge_scatter(d2, nj, bf=False):
    cc = nj * _W
    ng = _RPTI // nj
    assert ng % 4 == 0 and _RPTI % nj == 0
    mesh = plsc.VectorSubcoreMesh(core_axis_name="c", subcore_axis_name="s")
    cp = pltpu.CompilerParams()
    if "needs_layout_passes" in pltpu.CompilerParams.__dataclass_fields__:
        cp = dataclasses.replace(cp, needs_layout_passes=False,
                                 use_tc_tiling_on_sc=False)

    ydt = jnp.bfloat16 if bf else jnp.float32
    scratch = [
        pltpu.VMEM((4, 3, nj, _W), jnp.int32),
    ]
    if bf:
        scratch.append(pltpu.VMEM((2, cc, d2), jnp.bfloat16))
    scratch += [
        pltpu.VMEM((2, cc, d2), jnp.float32),
        pltpu.VMEM_SHARED((_NPAD, d2), jnp.float32),
    ] + [pltpu.SemaphoreType.DMA] * 8

    @functools.partial(
        pl.kernel,
        out_type=jax.ShapeDtypeStruct((_NCORE, _NPAD, d2), jnp.float32),
        mesh=mesh,
        compiler_params=cp,
        scratch_types=scratch,
    )
    def k(y_hbm, ed_hbm, out_hbm, *rest):
        edb = rest[0]
        rows_g = rest[1]
        rows = rest[2] if bf else rest[1]
        acc = rest[2 + bf]
        sem_i = rest[3 + bf:7 + bf]
        sem_g = rest[7 + bf:9 + bf]
        sem_s = rest[9 + bf:11 + bf]
        c = lax.axis_index("c")
        s = lax.axis_index("s")

        def issue_idx(slot, g):
            rb = s * _RPTI + g * nj
            for tt in range(3):
                pltpu.async_copy(ed_hbm.at[tt, pl.ds(rb, nj)],
                                 edb.at[slot, tt], sem_i[slot])

        def wait_idx(slot):
            pltpu.make_async_copy(ed_hbm.at[pl.ds(0, 3), pl.ds(0, nj)],
                                  edb.at[slot], sem_i[slot]).wait()

        def drain_gathers(b):
            pltpu.make_async_copy(y_hbm.at[c].at[pl.ds(0, cc)],
                                  rows_g.at[b], sem_g[b]).wait()

        def drain_scatters(b):
            pltpu.make_async_copy(rows.at[b], acc.at[pl.ds(0, cc)],
                                  sem_s[b]).wait()

        def issue_gathers(b, slot):
            for j in range(nj):
                pltpu.async_copy(y_hbm.at[c].at[edb.at[slot, 0, j]],
                                 rows_g.at[b, pl.ds(j * _W, _W)], sem_g[b])

        def scale(b, slot):
            for j in range(nj):

                @pl.loop(0, _W // _LANES)
                def _(e16):
                    cv = plsc.bitcast(edb[slot, 2, j, pl.ds(e16 * _LANES, _LANES)],
                                      jnp.float32)
                    base = j * _W + e16 * _LANES
                    for jj in range(_LANES):
                        sp = lax.gather(
                            cv, jnp.full((_LANES, 1), jj, jnp.int32),
                            lax.GatherDimensionNumbers(
                                offset_dims=(), collapsed_slice_dims=(0,),
                                start_index_map=(0,)),
                            (1,),
                            mode=lax.GatherScatterMode.PROMISE_IN_BOUNDS)
                        e = base + jj
                        if bf:
                            rv = rows_g[b, e, :]
                            vi = plsc.bitcast(rv, jnp.int32)
                            lo = plsc.bitcast(vi << 16, jnp.float32)
                            hi = plsc.bitcast(vi & jnp.int32(-65536), jnp.float32)
                            rows[b, e, pl.ds(0, _LANES)] = lo * sp
                            rows[b, e, pl.ds(_LANES, _LANES)] = hi * sp
                        else:
                            for kk in range(d2 // _LANES):
                                sl = (e, pl.ds(kk * _LANES, _LANES))
                                rows[(b,) + sl] = rows[(b,) + sl] * sp

        def issue_scatters(b, slot):
            for j in range(nj):
                pltpu.async_copy(rows.at[b, pl.ds(j * _W, _W)],
                                 acc.at[edb.at[slot, 1, j]], sem_s[b], add=True)

        issue_idx(0, 0)
        issue_idx(1, 1)

        @pl.loop(0, _ZR)
        def _(r):
            for kk in range(d2 // _LANES):
                rows[0, r, pl.ds(kk * _LANES, _LANES)] = (
                    jnp.zeros((_LANES,), jnp.float32))

        @pl.loop(0, _RPT // _ZR)
        def _(b):
            pltpu.sync_copy(rows.at[0, pl.ds(0, _ZR)],
                            acc.at[pl.ds(s * _RPT + b * _ZR, _ZR)])

        plsc.subcore_barrier()

        @pl.loop(0, ng, step=4)
        def _(g0):
            for b in range(4):
                g = g0 + b
                br = b % 2
                po = (b + 3) % 4

                @pl.when(g >= 2)
                def _():
                    drain_scatters(br)

                @pl.when(g < ng - 2)
                def _():
                    issue_idx((b + 2) % 4, g + 2)

                wait_idx(b)
                issue_gathers(br, b)

                @pl.when(g >= 1)
                def _():
                    drain_gathers(1 - br)
                    scale(1 - br, po)
                    issue_scatters(1 - br, po)

        drain_gathers(1)
        scale(1, 3)
        issue_scatters(1, 3)
        drain_scatters(0)
        drain_scatters(1)

        plsc.subcore_barrier()
        pltpu.sync_copy(acc.at[pl.ds(s * _RPT, _RPT)],
                        out_hbm.at[c].at[pl.ds(s * _RPT, _RPT)])

    return k


def _interleave_perm(width):
    half = width // 2
    perm = []
    for c in range(width // 32):
        base = c * 32
        for pp in range(16):
            perm += [base + pp, base + 16 + pp]
    return perm


_BN = 2000


def _edata_tc(w2d, s2d, src2d, dst2d):
    rows = w2d.shape[0]
    rb = 128
    nblk = _ROWS // rb

    def body(w_ref, s_ref, src_ref, dst_ref, o_ref):
        i = pl.program_id(0)
        rowid = i * rb + lax.broadcasted_iota(jnp.int32, (rb, 128), 0)
        valid = rowid < rows
        zi = jnp.zeros((rb, 128), jnp.int32)
        o_ref[0, ...] = jnp.where(valid, src_ref[...], zi)
        o_ref[1, ...] = jnp.where(valid, dst_ref[...], zi)
        cf = lax.bitcast_convert_type(w_ref[...] * s_ref[...], jnp.int32)
        o_ref[2, ...] = jnp.where(valid, cf, zi)

    imap = lambda i: (jnp.minimum(i, (rows - 1) // rb), 0)
    return pl.pallas_call(
        body,
        grid=(nblk,),
        in_specs=[pl.BlockSpec((rb, 128), imap)] * 4,
        out_specs=pl.BlockSpec((3, rb, 128), lambda i: (0, i, 0)),
        out_shape=jax.ShapeDtypeStruct((3, _ROWS, 128), jnp.int32),
    )(w2d, s2d, src2d, dst2d)


def _pre_tc(x, wca, wcb, wl, bl):
    hid = wl.shape[1]
    h2 = hid // 2

    def body(x_ref, wca_ref, wcb_ref, wl_ref, bl_ref, y_ref, lin_ref):
        xb = x_ref[...]
        wc = wca_ref[...] + wcb_ref[...]
        y = jnp.dot(xb, wc, preferred_element_type=jnp.float32)
        y = y.astype(jnp.bfloat16)
        y_ref[0, ...] = y[:, :h2]
        y_ref[1, ...] = y[:, h2:]
        lin_ref[...] = (jnp.dot(xb, wl_ref[...],
                                preferred_element_type=jnp.float32) + bl_ref[...])

    grid = (_N // _BN,)
    ind = x.shape[1]
    return pl.pallas_call(
        body,
        grid=grid,
        in_specs=[
            pl.BlockSpec((_BN, ind), lambda i: (i, 0)),
            pl.BlockSpec((ind, hid), lambda i: (0, 0)),
            pl.BlockSpec((ind, hid), lambda i: (0, 0)),
            pl.BlockSpec((ind, hid), lambda i: (0, 0)),
            pl.BlockSpec((1, hid), lambda i: (0, 0)),
        ],
        out_specs=[
            pl.BlockSpec((2, _BN, h2), lambda i: (0, i, 0)),
            pl.BlockSpec((_BN, hid), lambda i: (i, 0)),
        ],
        out_shape=[
            jax.ShapeDtypeStruct((2, _N, h2), jnp.bfloat16),
            jax.ShapeDtypeStruct((_N, hid), jnp.float32),
        ],
    )(x, wca, wcb, wl, bl)


def _mid_tc(agg, lin1, bc1, wca, wcb, wl2, bl2):
    hid = lin1.shape[1]
    bot = wl2.shape[1]
    b2 = bot // 2

    def body(a_ref, l_ref, bc_ref, wca_ref, wcb_ref, wl_ref, bl_ref,
             y_ref, lin_ref):
        a = jnp.concatenate([a_ref[0], a_ref[1]], axis=1)
        h = jnp.maximum(a + bc_ref[...] + l_ref[...], 0.0)
        wc = wca_ref[...] + wcb_ref[...]
        y = jnp.dot(h, wc, preferred_element_type=jnp.float32)
        y_ref[0, ...] = y[:, :b2]
        y_ref[1, ...] = y[:, b2:]
        lin_ref[...] = (jnp.dot(h, wl_ref[...],
                                preferred_element_type=jnp.float32) + bl_ref[...])

    grid = (_N // _BN,)
    return pl.pallas_call(
        body,
        grid=grid,
        in_specs=[
            pl.BlockSpec((2, _BN, hid // 2), lambda i: (0, i, 0)),
            pl.BlockSpec((_BN, hid), lambda i: (i, 0)),
            pl.BlockSpec((1, hid), lambda i: (0, 0)),
            pl.BlockSpec((hid, bot), lambda i: (0, 0)),
            pl.BlockSpec((hid, bot), lambda i: (0, 0)),
            pl.BlockSpec((hid, bot), lambda i: (0, 0)),
            pl.BlockSpec((1, bot), lambda i: (0, 0)),
        ],
        out_specs=[
            pl.BlockSpec((2, _BN, b2), lambda i: (0, i, 0)),
            pl.BlockSpec((_BN, bot), lambda i: (i, 0)),
        ],
        out_shape=[
            jax.ShapeDtypeStruct((2, _N, b2), jnp.float32),
            jax.ShapeDtypeStruct((_N, bot), jnp.float32),
        ],
    )(agg, lin1, bc1, wca, wcb, wl2, bl2)


def _final_tc(agg, lin2, bc2):
    bot = lin2.shape[1]

    def body(a_ref, l_ref, bc_ref, o_ref):
        a = jnp.concatenate([a_ref[0], a_ref[1]], axis=1)
        o_ref[...] = jnp.maximum(a + bc_ref[...] + l_ref[...], 0.0)

    grid = (_N // _BN,)
    return pl.pallas_call(
        body,
        grid=grid,
        in_specs=[
            pl.BlockSpec((2, _BN, bot // 2), lambda i: (0, i, 0)),
            pl.BlockSpec((_BN, bot), lambda i: (i, 0)),
            pl.BlockSpec((1, bot), lambda i: (0, 0)),
        ],
        out_specs=pl.BlockSpec((_BN, bot), lambda i: (i, 0)),
        out_shape=jax.ShapeDtypeStruct((_N, bot), jnp.float32),
    )(agg, lin2, bc2)


_scatter64 = _make_edge_scatter(32, 2, bf=True)
_scatter32 = _make_edge_scatter(16, 10)


def kernel(x, edge_index, weight, sim, Wc1, bc1, Wl1, bl1, Wc2, bc2, Wl2, bl2):
    src2d = edge_index[0].reshape(_E // _W, _W)
    dst2d = edge_index[1].reshape(_E // _W, _W)
    w2d = weight.reshape(_E // _W, _W)
    s2d = sim.reshape(_E // _W, _W)
    ed = _edata_tc(w2d, s2d, src2d, dst2d)

    pm = jnp.asarray(_interleave_perm(64), jnp.int32)
    y1, lin1 = _pre_tc(x, Wc1[0, :, :, 0][:, pm], Wc1[0, :, :, 1][:, pm], Wl1,
                       bl1.reshape(1, -1))
    agg1 = _scatter64(y1, ed)
    y2, lin2 = _mid_tc(agg1, lin1, bc1.reshape(1, -1),
                       Wc2[0, :, :, 0], Wc2[0, :, :, 1], Wl2,
                       bl2.reshape(1, -1))
    agg2 = _scatter32(y2, ed)
    return _final_tc(agg2, lin2, bc2.reshape(1, -1))


# speedup vs baseline: 1.0254x; 1.0254x over previous
import dataclasses
import functools

import jax
import jax.numpy as jnp
from jax import lax
from jax.experimental import pallas as pl
from jax.experimental.pallas import tpu as pltpu
from jax.experimental.pallas import tpu_sc as plsc

_N = 50000
_E = 800000
_NCORE = 2
_NSUB = 16
_LANES = 16
_W = 128
_ROWS = 6400
_EPAD = _ROWS * _W
_RPTI = _ROWS // _NSUB
_EPT = _RPTI * _W
_NPAD = 50048
_RPT = _NPAD // _NSUB
_ZR = 184


def _make_edge_scatter(d2, nj):
    cc = nj * _W
    ng = _RPTI // nj
    assert ng % 4 == 0 and _RPTI % nj == 0
    mesh = plsc.VectorSubcoreMesh(core_axis_name="c", subcore_axis_name="s")
    cp = pltpu.CompilerParams()
    if "needs_layout_passes" in pltpu.CompilerParams.__dataclass_fields__:
        cp = dataclasses.replace(cp, needs_layout_passes=False,
                                 use_tc_tiling_on_sc=False)

    @functools.partial(
        pl.kernel,
        out_type=jax.ShapeDtypeStruct((_NCORE, _NPAD, d2), jnp.float32),
        mesh=mesh,
        compiler_params=cp,
        scratch_types=[
            pltpu.VMEM((4, 3, nj, _W), jnp.int32),
            pltpu.VMEM((2, cc, d2), jnp.float32),
            pltpu.VMEM((_ZR, d2), jnp.float32),
            pltpu.VMEM_SHARED((_NPAD, d2), jnp.float32),
            pltpu.SemaphoreType.DMA,
            pltpu.SemaphoreType.DMA,
            pltpu.SemaphoreType.DMA,
            pltpu.SemaphoreType.DMA,
            pltpu.SemaphoreType.DMA,
            pltpu.SemaphoreType.DMA,
            pltpu.SemaphoreType.DMA,
            pltpu.SemaphoreType.DMA,
        ],
    )
    def k(y_hbm, ed_hbm, out_hbm, edb, rows, zb, acc,
          sem_i0, sem_i1, sem_i2, sem_i3, sem_g0, sem_g1, sem_s0, sem_s1):
        c = lax.axis_index("c")
        s = lax.axis_index("s")
        sem_i = (sem_i0, sem_i1, sem_i2, sem_i3)
        sem_g = (sem_g0, sem_g1)
        sem_s = (sem_s0, sem_s1)

        def issue_idx(slot, g):
            rb = s * _RPTI + g * nj
            for tt in range(3):
                pltpu.async_copy(ed_hbm.at[tt, pl.ds(rb, nj)],
                                 edb.at[slot, tt], sem_i[slot])

        def wait_idx(slot):
            pltpu.make_async_copy(ed_hbm.at[pl.ds(0, 3), pl.ds(0, nj)],
                                  edb.at[slot], sem_i[slot]).wait()

        def drain_gathers(b):
            pltpu.make_async_copy(y_hbm.at[c].at[pl.ds(0, cc)],
                                  rows.at[b], sem_g[b]).wait()

        def drain_scatters(b):
            pltpu.make_async_copy(rows.at[b], acc.at[pl.ds(0, cc)],
                                  sem_s[b]).wait()

        def issue_gathers(b, slot):
            for j in range(nj):
                pltpu.async_copy(y_hbm.at[c].at[edb.at[slot, 0, j]],
                                 rows.at[b, pl.ds(j * _W, _W)], sem_g[b])

        def scale(b, slot):
            for j in range(nj):

                @pl.loop(0, _W // _LANES)
                def _(e16):
                    cv = plsc.bitcast(edb[slot, 2, j, pl.ds(e16 * _LANES, _LANES)],
                                      jnp.float32)
                    base = j * _W + e16 * _LANES
                    for jj in range(_LANES):
                        sp = lax.gather(
                            cv, jnp.full((_LANES, 1), jj, jnp.int32),
                            lax.GatherDimensionNumbers(
                                offset_dims=(), collapsed_slice_dims=(0,),
                                start_index_map=(0,)),
                            (1,),
                            mode=lax.GatherScatterMode.PROMISE_IN_BOUNDS)
                        for kk in range(d2 // _LANES):
                            sl = (base + jj, pl.ds(kk * _LANES, _LANES))
                            rows[(b,) + sl] = rows[(b,) + sl] * sp

        def issue_scatters(b, slot):
            for j in range(nj):
                pltpu.async_copy(rows.at[b, pl.ds(j * _W, _W)],
                                 acc.at[edb.at[slot, 1, j]], sem_s[b], add=True)

        issue_idx(0, 0)
        issue_idx(1, 1)

        @pl.loop(0, _ZR)
        def _(r):
            for kk in range(d2 // _LANES):
                zb[r, pl.ds(kk * _LANES, _LANES)] = jnp.zeros((_LANES,), jnp.float32)

        @pl.loop(0, _RPT // _ZR)
        def _(b):
            pltpu.sync_copy(zb, acc.at[pl.ds(s * _RPT + b * _ZR, _ZR)])

        plsc.subcore_barrier()

        @pl.loop(0, ng, step=4)
        def _(g0):
            for b in range(4):
                g = g0 + b
                br = b % 2
                po = (b + 3) % 4

                @pl.when(g >= 2)
                def _():
                    drain_scatters(br)

                @pl.when(g < ng - 2)
                def _():
                    issue_idx((b + 2) % 4, g + 2)

                wait_idx(b)
                issue_gathers(br, b)

                @pl.when(g >= 1)
                def _():
                    drain_gathers(1 - br)
                    scale(1 - br, po)
                    issue_scatters(1 - br, po)

        drain_gathers(1)
        scale(1, 3)
        issue_scatters(1, 3)
        drain_scatters(0)
        drain_scatters(1)

        plsc.subcore_barrier()
        pltpu.sync_copy(acc.at[pl.ds(s * _RPT, _RPT)],
                        out_hbm.at[c].at[pl.ds(s * _RPT, _RPT)])

    return k


_BN = 2000


def _edata_tc(w2d, s2d, src2d, dst2d):
    rows = w2d.shape[0]
    rb = 128
    nblk = _ROWS // rb

    def body(w_ref, s_ref, src_ref, dst_ref, o_ref):
        i = pl.program_id(0)
        rowid = i * rb + lax.broadcasted_iota(jnp.int32, (rb, 128), 0)
        valid = rowid < rows
        zi = jnp.zeros((rb, 128), jnp.int32)
        o_ref[0, ...] = jnp.where(valid, src_ref[...], zi)
        o_ref[1, ...] = jnp.where(valid, dst_ref[...], zi)
        cf = lax.bitcast_convert_type(w_ref[...] * s_ref[...], jnp.int32)
        o_ref[2, ...] = jnp.where(valid, cf, zi)

    imap = lambda i: (jnp.minimum(i, (rows - 1) // rb), 0)
    return pl.pallas_call(
        body,
        grid=(nblk,),
        in_specs=[pl.BlockSpec((rb, 128), imap)] * 4,
        out_specs=pl.BlockSpec((3, rb, 128), lambda i: (0, i, 0)),
        out_shape=jax.ShapeDtypeStruct((3, _ROWS, 128), jnp.int32),
    )(w2d, s2d, src2d, dst2d)


def _pre_tc(x, wca, wcb, wl, bl):
    hid = wl.shape[1]
    h2 = hid // 2

    def body(x_ref, wca_ref, wcb_ref, wl_ref, bl_ref, y_ref, lin_ref):
        xb = x_ref[...]
        wc = wca_ref[...] + wcb_ref[...]
        y = jnp.dot(xb, wc, preferred_element_type=jnp.float32)
        y_ref[0, ...] = y[:, :h2]
        y_ref[1, ...] = y[:, h2:]
        lin_ref[...] = (jnp.dot(xb, wl_ref[...],
                                preferred_element_type=jnp.float32) + bl_ref[...])

    grid = (_N // _BN,)
    ind = x.shape[1]
    return pl.pallas_call(
        body,
        grid=grid,
        in_specs=[
            pl.BlockSpec((_BN, ind), lambda i: (i, 0)),
            pl.BlockSpec((ind, hid), lambda i: (0, 0)),
            pl.BlockSpec((ind, hid), lambda i: (0, 0)),
            pl.BlockSpec((ind, hid), lambda i: (0, 0)),
            pl.BlockSpec((1, hid), lambda i: (0, 0)),
        ],
        out_specs=[
            pl.BlockSpec((2, _BN, h2), lambda i: (0, i, 0)),
            pl.BlockSpec((_BN, hid), lambda i: (i, 0)),
        ],
        out_shape=[
            jax.ShapeDtypeStruct((2, _N, h2), jnp.float32),
            jax.ShapeDtypeStruct((_N, hid), jnp.float32),
        ],
    )(x, wca, wcb, wl, bl)


def _mid_tc(agg, lin1, bc1, wca, wcb, wl2, bl2):
    hid = lin1.shape[1]
    bot = wl2.shape[1]
    b2 = bot // 2

    def body(a_ref, l_ref, bc_ref, wca_ref, wcb_ref, wl_ref, bl_ref,
             y_ref, lin_ref):
        a = jnp.concatenate([a_ref[0], a_ref[1]], axis=1)
        h = jnp.maximum(a + bc_ref[...] + l_ref[...], 0.0)
        wc = wca_ref[...] + wcb_ref[...]
        y = jnp.dot(h, wc, preferred_element_type=jnp.float32)
        y_ref[0, ...] = y[:, :b2]
        y_ref[1, ...] = y[:, b2:]
        lin_ref[...] = (jnp.dot(h, wl_ref[...],
                                preferred_element_type=jnp.float32) + bl_ref[...])

    grid = (_N // _BN,)
    return pl.pallas_call(
        body,
        grid=grid,
        in_specs=[
            pl.BlockSpec((2, _BN, hid // 2), lambda i: (0, i, 0)),
            pl.BlockSpec((_BN, hid), lambda i: (i, 0)),
            pl.BlockSpec((1, hid), lambda i: (0, 0)),
            pl.BlockSpec((hid, bot), lambda i: (0, 0)),
            pl.BlockSpec((hid, bot), lambda i: (0, 0)),
            pl.BlockSpec((hid, bot), lambda i: (0, 0)),
            pl.BlockSpec((1, bot), lambda i: (0, 0)),
        ],
        out_specs=[
            pl.BlockSpec((2, _BN, b2), lambda i: (0, i, 0)),
            pl.BlockSpec((_BN, bot), lambda i: (i, 0)),
        ],
        out_shape=[
            jax.ShapeDtypeStruct((2, _N, b2), jnp.float32),
            jax.ShapeDtypeStruct((_N, bot), jnp.float32),
        ],
    )(agg, lin1, bc1, wca, wcb, wl2, bl2)


def _final_tc(agg, lin2, bc2):
    bot = lin2.shape[1]

    def body(a_ref, l_ref, bc_ref, o_ref):
        a = jnp.concatenate([a_ref[0], a_ref[1]], axis=1)
        o_ref[...] = jnp.maximum(a + bc_ref[...] + l_ref[...], 0.0)

    grid = (_N // _BN,)
    return pl.pallas_call(
        body,
        grid=grid,
        in_specs=[
            pl.BlockSpec((2, _BN, bot // 2), lambda i: (0, i, 0)),
            pl.BlockSpec((_BN, bot), lambda i: (i, 0)),
            pl.BlockSpec((1, bot), lambda i: (0, 0)),
        ],
        out_specs=pl.BlockSpec((_BN, bot), lambda i: (i, 0)),
        out_shape=jax.ShapeDtypeStruct((_N, bot), jnp.float32),
    )(agg, lin2, bc2)


_scatter64 = _make_edge_scatter(32, 2)
_scatter32 = _make_edge_scatter(16, 10)


def kernel(x, edge_index, weight, sim, Wc1, bc1, Wl1, bl1, Wc2, bc2, Wl2, bl2):
    src2d = edge_index[0].reshape(_E // _W, _W)
    dst2d = edge_index[1].reshape(_E // _W, _W)
    w2d = weight.reshape(_E // _W, _W)
    s2d = sim.reshape(_E // _W, _W)
    ed = _edata_tc(w2d, s2d, src2d, dst2d)

    y1, lin1 = _pre_tc(x, Wc1[0, :, :, 0], Wc1[0, :, :, 1], Wl1,
                       bl1.reshape(1, -1))
    agg1 = _scatter64(y1, ed)
    y2, lin2 = _mid_tc(agg1, lin1, bc1.reshape(1, -1),
                       Wc2[0, :, :, 0], Wc2[0, :, :, 1], Wl2,
                       bl2.reshape(1, -1))
    agg2 = _scatter32(y2, ed)
    return _final_tc(agg2, lin2, bc2.reshape(1, -1))


# speedup vs baseline: 1.0441x; 1.0182x over previous
import dataclasses
import functools

import jax
import jax.numpy as jnp
from jax import lax
from jax.experimental import pallas as pl
from jax.experimental.pallas import tpu as pltpu
from jax.experimental.pallas import tpu_sc as plsc

_N = 50000
_E = 800000
_NCORE = 2
_NSUB = 16
_LANES = 16
_W = 128
_ROWS = 6400
_EPAD = _ROWS * _W
_RPTI = _ROWS // _NSUB
_EPT = _RPTI * _W
_NPAD = 50048
_RPT = _NPAD // _NSUB
_ZR = 184


def _make_edge_scatter(d2, nj):
    cc = nj * _W
    ng = _RPTI // nj
    assert ng % 4 == 0 and _RPTI % nj == 0
    mesh = plsc.VectorSubcoreMesh(core_axis_name="c", subcore_axis_name="s")
    cp = pltpu.CompilerParams()
    if "needs_layout_passes" in pltpu.CompilerParams.__dataclass_fields__:
        cp = dataclasses.replace(cp, needs_layout_passes=False,
                                 use_tc_tiling_on_sc=False)

    @functools.partial(
        pl.kernel,
        out_type=jax.ShapeDtypeStruct((_NCORE, _NPAD, d2), jnp.float32),
        mesh=mesh,
        compiler_params=cp,
        scratch_types=[
            pltpu.VMEM((4, 3, nj, _W), jnp.int32),
            pltpu.VMEM((2, cc, d2), jnp.float32),
            pltpu.VMEM((_ZR, d2), jnp.float32),
            pltpu.VMEM_SHARED((_NPAD, d2), jnp.float32),
            pltpu.SemaphoreType.DMA,
            pltpu.SemaphoreType.DMA,
            pltpu.SemaphoreType.DMA,
            pltpu.SemaphoreType.DMA,
            pltpu.SemaphoreType.DMA,
            pltpu.SemaphoreType.DMA,
            pltpu.SemaphoreType.DMA,
            pltpu.SemaphoreType.DMA,
        ],
    )
    def k(y_hbm, ed_hbm, out_hbm, edb, rows, zb, acc,
          sem_i0, sem_i1, sem_i2, sem_i3, sem_g0, sem_g1, sem_s0, sem_s1):
        c = lax.axis_index("c")
        s = lax.axis_index("s")
        sem_i = (sem_i0, sem_i1, sem_i2, sem_i3)
        sem_g = (sem_g0, sem_g1)
        sem_s = (sem_s0, sem_s1)

        def issue_idx(slot, g):
            rb = s * _RPTI + g * nj
            for tt in range(3):
                pltpu.async_copy(ed_hbm.at[tt, pl.ds(rb, nj)],
                                 edb.at[slot, tt], sem_i[slot])

        def wait_idx(slot):
            pltpu.make_async_copy(ed_hbm.at[pl.ds(0, 3), pl.ds(0, nj)],
                                  edb.at[slot], sem_i[slot]).wait()

        def drain_gathers(b):
            pltpu.make_async_copy(y_hbm.at[c].at[pl.ds(0, cc)],
                                  rows.at[b], sem_g[b]).wait()

        def drain_scatters(b):
            pltpu.make_async_copy(rows.at[b], acc.at[pl.ds(0, cc)],
                                  sem_s[b]).wait()

        def issue_gathers(b, slot):
            for j in range(nj):
                pltpu.async_copy(y_hbm.at[c].at[edb.at[slot, 0, j]],
                                 rows.at[b, pl.ds(j * _W, _W)], sem_g[b])

        def scale(b, slot):
            for j in range(nj):

                @pl.loop(0, _W // _LANES)
                def _(e16):
                    cv = plsc.bitcast(edb[slot, 2, j, pl.ds(e16 * _LANES, _LANES)],
                                      jnp.float32)
                    base = j * _W + e16 * _LANES
                    for jj in range(_LANES):
                        sp = lax.gather(
                            cv, jnp.full((_LANES, 1), jj, jnp.int32),
                            lax.GatherDimensionNumbers(
                                offset_dims=(), collapsed_slice_dims=(0,),
                                start_index_map=(0,)),
                            (1,),
                            mode=lax.GatherScatterMode.PROMISE_IN_BOUNDS)
                        for kk in range(d2 // _LANES):
                            sl = (base + jj, pl.ds(kk * _LANES, _LANES))
                            rows[(b,) + sl] = rows[(b,) + sl] * sp

        def issue_scatters(b, slot):
            for j in range(nj):
                pltpu.async_copy(rows.at[b, pl.ds(j * _W, _W)],
                                 acc.at[edb.at[slot, 1, j]], sem_s[b], add=True)

        issue_idx(0, 0)
        issue_idx(1, 1)

        @pl.loop(0, _ZR)
        def _(r):
            for kk in range(d2 // _LANES):
                zb[r, pl.ds(kk * _LANES, _LANES)] = jnp.zeros((_LANES,), jnp.float32)

        @pl.loop(0, _RPT // _ZR)
        def _(b):
            pltpu.sync_copy(zb, acc.at[pl.ds(s * _RPT + b * _ZR, _ZR)])

        plsc.subcore_barrier()

        @pl.loop(0, ng, step=4)
        def _(g0):
            for b in range(4):
                g = g0 + b
                br = b % 2
                po = (b + 3) % 4

                @pl.when(g >= 2)
                def _():
                    drain_scatters(br)

                @pl.when(g < ng - 2)
                def _():
                    issue_idx((b + 2) % 4, g + 2)

                wait_idx(b)
                issue_gathers(br, b)

                @pl.when(g >= 1)
                def _():
                    drain_gathers(1 - br)
                    scale(1 - br, po)
                    issue_scatters(1 - br, po)

        drain_gathers(1)
        scale(1, 3)
        issue_scatters(1, 3)
        drain_scatters(0)
        drain_scatters(1)

        plsc.subcore_barrier()
        pltpu.sync_copy(acc.at[pl.ds(s * _RPT, _RPT)],
                        out_hbm.at[c].at[pl.ds(s * _RPT, _RPT)])

    return k


_BN = 2000


def _edata_tc(w1, s1, src1, dst1):
    er = _E // _W

    def body(w_ref, s_ref, src_ref, dst_ref, o_ref):
        o_ref[0, pl.ds(0, er), :] = src_ref[...].reshape(er, _W)
        o_ref[1, pl.ds(0, er), :] = dst_ref[...].reshape(er, _W)
        cf = lax.bitcast_convert_type(w_ref[...] * s_ref[...], jnp.int32)
        o_ref[2, pl.ds(0, er), :] = cf.reshape(er, _W)
        zi = jnp.zeros((_ROWS - er, _W), jnp.int32)
        for tt in range(3):
            o_ref[tt, pl.ds(er, _ROWS - er), :] = zi

    return pl.pallas_call(
        body,
        out_shape=jax.ShapeDtypeStruct((3, _ROWS, 128), jnp.int32),
    )(w1, s1, src1, dst1)


def _pre_tc(x, wca, wcb, wl, bl):
    hid = wl.shape[1]
    h2 = hid // 2

    def body(x_ref, wca_ref, wcb_ref, wl_ref, bl_ref, y_ref, lin_ref):
        xb = x_ref[...]
        wc = wca_ref[...] + wcb_ref[...]
        y = jnp.dot(xb, wc, preferred_element_type=jnp.float32)
        y_ref[0, ...] = y[:, :h2]
        y_ref[1, ...] = y[:, h2:]
        lin_ref[...] = (jnp.dot(xb, wl_ref[...],
                                preferred_element_type=jnp.float32) + bl_ref[...])

    grid = (_N // _BN,)
    ind = x.shape[1]
    return pl.pallas_call(
        body,
        grid=grid,
        in_specs=[
            pl.BlockSpec((_BN, ind), lambda i: (i, 0)),
            pl.BlockSpec((ind, hid), lambda i: (0, 0)),
            pl.BlockSpec((ind, hid), lambda i: (0, 0)),
            pl.BlockSpec((ind, hid), lambda i: (0, 0)),
            pl.BlockSpec((1, hid), lambda i: (0, 0)),
        ],
        out_specs=[
            pl.BlockSpec((2, _BN, h2), lambda i: (0, i, 0)),
            pl.BlockSpec((_BN, hid), lambda i: (i, 0)),
        ],
        out_shape=[
            jax.ShapeDtypeStruct((2, _N, h2), jnp.float32),
            jax.ShapeDtypeStruct((_N, hid), jnp.float32),
        ],
    )(x, wca, wcb, wl, bl)


def _mid_tc(agg, lin1, bc1, wca, wcb, wl2, bl2):
    hid = lin1.shape[1]
    bot = wl2.shape[1]
    b2 = bot // 2

    def body(a_ref, l_ref, bc_ref, wca_ref, wcb_ref, wl_ref, bl_ref,
             y_ref, lin_ref):
        a = jnp.concatenate([a_ref[0], a_ref[1]], axis=1)
        h = jnp.maximum(a + bc_ref[...] + l_ref[...], 0.0)
        wc = wca_ref[...] + wcb_ref[...]
        y = jnp.dot(h, wc, preferred_element_type=jnp.float32)
        y_ref[0, ...] = y[:, :b2]
        y_ref[1, ...] = y[:, b2:]
        lin_ref[...] = (jnp.dot(h, wl_ref[...],
                                preferred_element_type=jnp.float32) + bl_ref[...])

    grid = (_N // _BN,)
    return pl.pallas_call(
        body,
        grid=grid,
        in_specs=[
            pl.BlockSpec((2, _BN, hid // 2), lambda i: (0, i, 0)),
            pl.BlockSpec((_BN, hid), lambda i: (i, 0)),
            pl.BlockSpec((1, hid), lambda i: (0, 0)),
            pl.BlockSpec((hid, bot), lambda i: (0, 0)),
            pl.BlockSpec((hid, bot), lambda i: (0, 0)),
            pl.BlockSpec((hid, bot), lambda i: (0, 0)),
            pl.BlockSpec((1, bot), lambda i: (0, 0)),
        ],
        out_specs=[
            pl.BlockSpec((2, _BN, b2), lambda i: (0, i, 0)),
            pl.BlockSpec((_BN, bot), lambda i: (i, 0)),
        ],
        out_shape=[
            jax.ShapeDtypeStruct((2, _N, b2), jnp.float32),
            jax.ShapeDtypeStruct((_N, bot), jnp.float32),
        ],
    )(agg, lin1, bc1, wca, wcb, wl2, bl2)


def _final_tc(agg, lin2, bc2):
    bot = lin2.shape[1]

    def body(a_ref, l_ref, bc_ref, o_ref):
        a = jnp.concatenate([a_ref[0], a_ref[1]], axis=1)
        o_ref[...] = jnp.maximum(a + bc_ref[...] + l_ref[...], 0.0)

    grid = (_N // _BN,)
    return pl.pallas_call(
        body,
        grid=grid,
        in_specs=[
            pl.BlockSpec((2, _BN, bot // 2), lambda i: (0, i, 0)),
            pl.BlockSpec((_BN, bot), lambda i: (i, 0)),
            pl.BlockSpec((1, bot), lambda i: (0, 0)),
        ],
        out_specs=pl.BlockSpec((_BN, bot), lambda i: (i, 0)),
        out_shape=jax.ShapeDtypeStruct((_N, bot), jnp.float32),
    )(agg, lin2, bc2)


_scatter64 = _make_edge_scatter(32, 2)
_scatter32 = _make_edge_scatter(16, 10)


def kernel(x, edge_index, weight, sim, Wc1, bc1, Wl1, bl1, Wc2, bc2, Wl2, bl2):
    src1 = edge_index[0]
    dst1 = edge_index[1]
    ed = _edata_tc(weight, sim, src1, dst1)

    y1, lin1 = _pre_tc(x, Wc1[0, :, :, 0], Wc1[0, :, :, 1], Wl1,
                       bl1.reshape(1, -1))
    agg1 = _scatter64(y1, ed)
    y2, lin2 = _mid_tc(agg1, lin1, bc1.reshape(1, -1),
                       Wc2[0, :, :, 0], Wc2[0, :, :, 1], Wl2,
                       bl2.reshape(1, -1))
    agg2 = _scatter32(y2, ed)
    return _final_tc(agg2, lin2, bc2.reshape(1, -1))
